# baseline (speedup 1.0000x reference)
import math

import jax
import jax.numpy as jnp
import numpy as np
from jax.experimental import pallas as pl
from jax.experimental.pallas import tpu as pltpu

_EPS = jnp.float32(0.0001)


def _fused_logits_body(gc_ref, qx_ref, kw0_ref, kb0_ref, kw1_ref, kb1_ref,
                       qfw_ref, qfb_ref, out_ref, *, scale):
    x = gc_ref[...]
    h = jnp.maximum(
        jnp.dot(x, kw0_ref[...], preferred_element_type=jnp.float32)
        + kb0_ref[...], 0.0)
    h = jnp.maximum(
        jnp.dot(h, kw1_ref[...], preferred_element_type=jnp.float32)
        + kb1_ref[...], 0.0)
    k = (jnp.dot(h, qfw_ref[...], preferred_element_type=jnp.float32)
         + qfb_ref[...])
    out_ref[...] = jnp.sum(qx_ref[...] * k, axis=1) / scale


def _fused_logits(g_candidates, qx, Kw0, Kb0, Kw1, Kb1, Qfw, Qfb):
    n, d = g_candidates.shape
    o = Qfw.shape[1]
    tile = 8192
    grid = (n // tile,)
    scale = np.float32(o ** 0.5)
    import functools
    body = functools.partial(_fused_logits_body, scale=scale)
    return pl.pallas_call(
        body,
        grid=grid,
        in_specs=[
            pl.BlockSpec((tile, d), lambda i: (i, 0)),
            pl.BlockSpec((tile, o), lambda i: (i, 0)),
            pl.BlockSpec(Kw0.shape, lambda i: (0, 0)),
            pl.BlockSpec(Kb0.shape, lambda i: (0,)),
            pl.BlockSpec(Kw1.shape, lambda i: (0, 0)),
            pl.BlockSpec(Kb1.shape, lambda i: (0,)),
            pl.BlockSpec(Qfw.shape, lambda i: (0, 0)),
            pl.BlockSpec(Qfb.shape, lambda i: (0,)),
        ],
        out_specs=pl.BlockSpec((tile,), lambda i: (i,)),
        out_shape=jax.ShapeDtypeStruct((n,), jnp.float32),
    )(g_candidates, qx, Kw0, Kb0, Kw1, Kb1, Qfw, Qfb)


def kernel(g, g_candidates, batch_idx, gumbel, Qw0, Qb0, Qw1, Qb1,
           Kw0, Kb0, Kw1, Kb1, Qfw, Qfb):
    B = g.shape[0]
    N = g_candidates.shape[0]
    d_k = Qfw.shape[1]
    relu = jax.nn.relu

    Q = relu(g @ Qw0 + Qb0)
    Q = relu(Q @ Qw1 + Qb1)
    Q = Q @ Qfw + Qfb

    Qx = jnp.take(Q, batch_idx, axis=0)
    logits = _fused_logits(g_candidates, Qx, Kw0, Kb0, Kw1, Kb1, Qfw, Qfb)

    lmax = jax.ops.segment_max(logits, batch_idx, num_segments=B)
    logits_s = logits - jnp.take(lmax, batch_idx, axis=0)
    e = jnp.exp(logits_s)
    lsum = jax.ops.segment_sum(e, batch_idx, num_segments=B) + _EPS
    probs = e / jnp.take(lsum, batch_idx, axis=0)

    logp = jnp.where(probs > _EPS, jnp.log(jnp.maximum(probs, 1e-30)),
                     -jnp.inf)
    keyed = logp + gumbel
    seg_max = jax.ops.segment_max(keyed, batch_idx, num_segments=B)
    is_max = keyed >= jnp.take(seg_max, batch_idx, axis=0)
    idx = jnp.arange(N, dtype=jnp.int32)
    shifted_actions = jax.ops.segment_min(jnp.where(is_max, idx, N),
                                          batch_idx, num_segments=B)
    shifted_actions = jnp.minimum(shifted_actions, N - 1)

    counts = jnp.bincount(batch_idx, length=B)
    offsets = jnp.concatenate(
        [jnp.zeros((1,), counts.dtype), jnp.cumsum(counts)[:-1]])
    actions = shifted_actions - offsets
    action_logprobs = jnp.log(jnp.maximum(
        jnp.take(probs, shifted_actions, axis=0), 1e-30))
    g_next_emb = jnp.take(g_candidates, shifted_actions, axis=0)
    return (g, g_next_emb, g_candidates, probs, action_logprobs, actions,
            shifted_actions)


# speedup vs baseline: 1.0719x; 1.0719x over previous
import functools

import jax
import jax.numpy as jnp
import numpy as np
from jax.experimental import pallas as pl

_EPS = jnp.float32(0.0001)


def _fused_logits_body(gc_ref, qx_ref, kw0_ref, kb0_ref, kw1_ref, kb1_ref,
                       qfw_ref, qfb_ref, out_ref, *, scale):
    x = gc_ref[...]
    h = jnp.maximum(
        jnp.dot(x, kw0_ref[...], preferred_element_type=jnp.float32)
        + kb0_ref[...], 0.0)
    h = jnp.maximum(
        jnp.dot(h, kw1_ref[...], preferred_element_type=jnp.float32)
        + kb1_ref[...], 0.0)
    k = (jnp.dot(h, qfw_ref[...], preferred_element_type=jnp.float32)
         + qfb_ref[...])
    out_ref[...] = jnp.sum(qx_ref[...] * k, axis=1) / scale


def _fused_logits(g_candidates, qx, Kw0, Kb0, Kw1, Kb1, Qfw, Qfb):
    n, d = g_candidates.shape
    o = Qfw.shape[1]
    tile = 8192
    grid = (n // tile,)
    scale = np.float32(o ** 0.5)
    body = functools.partial(_fused_logits_body, scale=scale)
    return pl.pallas_call(
        body,
        grid=grid,
        in_specs=[
            pl.BlockSpec((tile, d), lambda i: (i, 0)),
            pl.BlockSpec((tile, o), lambda i: (i, 0)),
            pl.BlockSpec(Kw0.shape, lambda i: (0, 0)),
            pl.BlockSpec(Kb0.shape, lambda i: (0,)),
            pl.BlockSpec(Kw1.shape, lambda i: (0, 0)),
            pl.BlockSpec(Kb1.shape, lambda i: (0,)),
            pl.BlockSpec(Qfw.shape, lambda i: (0, 0)),
            pl.BlockSpec(Qfb.shape, lambda i: (0,)),
        ],
        out_specs=pl.BlockSpec((tile,), lambda i: (i,)),
        out_shape=jax.ShapeDtypeStruct((n,), jnp.float32),
    )(g_candidates, qx, Kw0, Kb0, Kw1, Kb1, Qfw, Qfb)


def kernel(g, g_candidates, batch_idx, gumbel, Qw0, Qb0, Qw1, Qb1,
           Kw0, Kb0, Kw1, Kb1, Qfw, Qfb):
    B = g.shape[0]
    N = g_candidates.shape[0]
    d_k = Qfw.shape[1]
    relu = jax.nn.relu

    Q = relu(g @ Qw0 + Qb0)
    Q = relu(Q @ Qw1 + Qb1)
    Q = Q @ Qfw + Qfb

    Qx = jnp.take(Q, batch_idx, axis=0, indices_are_sorted=True)
    logits = _fused_logits(g_candidates, Qx, Kw0, Kb0, Kw1, Kb1, Qfw, Qfb)

    lmax = jax.ops.segment_max(logits, batch_idx, num_segments=B,
                               indices_are_sorted=True)
    logits_s = logits - jnp.take(lmax, batch_idx, axis=0,
                                 indices_are_sorted=True)
    e = jnp.exp(logits_s)
    lsum = jax.ops.segment_sum(e, batch_idx, num_segments=B,
                               indices_are_sorted=True) + _EPS
    probs = e / jnp.take(lsum, batch_idx, axis=0, indices_are_sorted=True)

    logp = jnp.where(probs > _EPS, jnp.log(jnp.maximum(probs, 1e-30)),
                     -jnp.inf)
    keyed = logp + gumbel
    seg_max = jax.ops.segment_max(keyed, batch_idx, num_segments=B,
                                  indices_are_sorted=True)
    is_max = keyed >= jnp.take(seg_max, batch_idx, axis=0,
                               indices_are_sorted=True)
    idx = jnp.arange(N, dtype=jnp.int32)
    shifted_actions = jax.ops.segment_min(jnp.where(is_max, idx, N),
                                          batch_idx, num_segments=B,
                                          indices_are_sorted=True)
    shifted_actions = jnp.minimum(shifted_actions, N - 1)

    counts = jax.ops.segment_sum(jnp.ones((N,), jnp.int32), batch_idx,
                                 num_segments=B, indices_are_sorted=True)
    offsets = jnp.concatenate(
        [jnp.zeros((1,), counts.dtype), jnp.cumsum(counts)[:-1]])
    actions = shifted_actions - offsets
    action_logprobs = jnp.log(jnp.maximum(
        jnp.take(probs, shifted_actions, axis=0), 1e-30))
    g_next_emb = jnp.take(g_candidates, shifted_actions, axis=0)
    return (g, g_next_emb, g_candidates, probs, action_logprobs, actions,
            shifted_actions)
